# Initial kernel scaffold; baseline (speedup 1.0000x reference)
import jax
import jax.numpy as jnp
from jax.experimental import pallas as pl


def kernel(x, edge_attr, edge_index, batch, W_in, b_in, att_l, att_r, W_edge, W_gout, b_gate, Wih0, Whh0, bih0, bhh0, Wa, att_src_a, att_dst_a, ba, Wih1, Whh1, bih1, bhh1, Wm, att_src_m, att_dst_m, bm, Wih2, Whh2, bih2, bhh2, W_out, b_out):
    raise NotImplementedError("write your pallas kernel here")


# speedup vs baseline: 8.7452x; 8.7452x over previous
import functools

import jax
import jax.numpy as jnp
from jax import lax
from jax.experimental import pallas as pl
from jax.experimental.pallas import tpu as pltpu
from jax.experimental.pallas import tpu_sc as plsc

H = 128
N = 10000
E = 320000
B = 512
NP = 10240
BP = 640
NC = 2
NS = 16
NW = NC * NS
EPW = E // NW
C = 80
CH = EPW // C
RPT_N = NP // NS
RPT_B = BP // NS
NPW = NP // NW
EPS = 1e-16

_mesh = plsc.VectorSubcoreMesh(core_axis_name="c", subcore_axis_name="s")
_f32 = jnp.float32


def _lrelu(v, s):
    return jnp.where(v >= 0, v, s * v)


def _zero_rows(buf, rows, cols):
    zv = jnp.zeros((16,), _f32)
    kk = cols // 16

    def body(i, _):
        r = i // kk
        k = i % kk
        buf[r, pl.ds(k * 16, 16)] = zv
        return 0

    lax.fori_loop(0, rows * kk, body, 0)


def _zero_shared(zsrc, sh, rows_per_tile, sid):
    zrows = zsrc.shape[0]
    base = sid * rows_per_tile
    done = 0
    while done < rows_per_tile:
        step = min(zrows, rows_per_tile - done)
        pltpu.sync_copy(zsrc.at[pl.ds(0, step)], sh.at[pl.ds(base + done, step)])
        done += step


def _writeback(sh, hbm, rows_per_tile, cid, sid):
    base = sid * rows_per_tile
    pltpu.sync_copy(sh.at[pl.ds(base, rows_per_tile)],
                    hbm.at[cid].at[pl.ds(base, rows_per_tile)])


def _scale_rows(Gbuf, W16, n_rows):
    def body(e, _):
        w = W16[e, pl.ds(0, 16)]
        for k in range(8):
            Gbuf[e, pl.ds(k * 16, 16)] = Gbuf[e, pl.ds(k * 16, 16)] * w
        return 0

    lax.fori_loop(0, n_rows, body, 0)


def _spread_w16(W16, ex16, g):
    for l in range(16):
        W16[g * 16 + l, pl.ds(0, 16)] = jnp.full((16,), ex16[l])


def _conv1_body(tl_hbm, eap_hbm, sr_hbm, gout_hbm, attl_hbm, src_hbm, dst_hbm,
                num_hbm, s_hbm,
                num_sh, s_sh,
                A, Bb, W16, src_c, dst_c, ar_c, attl_v, sem):
    cid = lax.axis_index("c")
    sid = lax.axis_index("s")
    wid = cid * NS + sid

    _zero_rows(A, C, H)
    _zero_rows(W16, C, 16)
    pltpu.sync_copy(attl_hbm, attl_v)
    _zero_shared(A, num_sh, RPT_N, sid)
    _zero_shared(W16, s_sh, RPT_N, sid)
    plsc.subcore_barrier()

    def chunk(i, _):
        base = wid * EPW + i * C
        pltpu.sync_copy(src_hbm.at[pl.ds(base, C)], src_c)
        pltpu.sync_copy(dst_hbm.at[pl.ds(base, C)], dst_c)
        pltpu.async_copy(tl_hbm.at[src_c], A, sem).wait()
        pltpu.sync_copy(eap_hbm.at[pl.ds(base, C)], Bb)
        pltpu.async_copy(sr_hbm.at[dst_c], ar_c, sem).wait()

        def e1(g, _):
            ar16 = ar_c[pl.ds(g * 16, 16)]
            for l in range(16):
                e = g * 16 + l
                acc = jnp.zeros((16,), _f32)
                for k in range(8):
                    v = A[e, pl.ds(k * 16, 16)] + Bb[e, pl.ds(k * 16, 16)]
                    v = _lrelu(v, 0.01)
                    acc = acc + v * attl_v[pl.ds(k * 16, 16)]
                aj = jnp.sum(acc)
                al = _lrelu(aj + ar16[l], 0.01)
                W16[e, pl.ds(0, 16)] = jnp.exp(jnp.full((16,), al))
            return 0

        lax.fori_loop(0, C // 16, e1, 0)

        pltpu.async_copy(gout_hbm.at[src_c], A, sem).wait()
        _scale_rows(A, W16, C)
        pltpu.sync_copy(A, num_sh.at[dst_c], add=True)
        pltpu.sync_copy(W16, s_sh.at[dst_c], add=True)
        return 0

    lax.fori_loop(0, CH, chunk, 0)
    plsc.subcore_barrier()
    _writeback(num_sh, num_hbm, RPT_N, cid, sid)
    _writeback(s_sh, s_hbm, RPT_N, cid, sid)


_conv1_call = functools.partial(
    pl.kernel, _conv1_body,
    out_type=(jax.ShapeDtypeStruct((NC, NP, H), _f32),
              jax.ShapeDtypeStruct((NC, NP, 16), _f32)),
    mesh=_mesh,
    compiler_params=pltpu.CompilerParams(needs_layout_passes=False, use_tc_tiling_on_sc=False),
    scratch_types=[
        pltpu.VMEM_SHARED((NP, H), _f32),
        pltpu.VMEM_SHARED((NP, 16), _f32),
        pltpu.VMEM((C, H), _f32),
        pltpu.VMEM((C, H), _f32),
        pltpu.VMEM((C, 16), _f32),
        pltpu.VMEM((C,), jnp.int32),
        pltpu.VMEM((C,), jnp.int32),
        pltpu.VMEM((C,), _f32),
        pltpu.VMEM((H,), _f32),
        pltpu.SemaphoreType.DMA,
    ],
)


def _conv2_body(xp_hbm, as_hbm, ad_hbm, src_hbm, dst_hbm,
                num_hbm, s_hbm,
                num_sh, s_sh,
                G, W16, src_c, dst_c, as_c, ad_c, sem):
    cid = lax.axis_index("c")
    sid = lax.axis_index("s")
    wid = cid * NS + sid

    _zero_rows(G, C, H)
    _zero_rows(W16, C, 16)
    _zero_shared(G, num_sh, RPT_N, sid)
    _zero_shared(W16, s_sh, RPT_N, sid)
    plsc.subcore_barrier()

    def chunk(i, _):
        base = wid * EPW + i * C
        pltpu.sync_copy(src_hbm.at[pl.ds(base, C)], src_c)
        pltpu.sync_copy(dst_hbm.at[pl.ds(base, C)], dst_c)
        pltpu.async_copy(as_hbm.at[src_c], as_c, sem).wait()
        pltpu.async_copy(ad_hbm.at[dst_c], ad_c, sem).wait()

        def e1(g, _):
            a = as_c[pl.ds(g * 16, 16)] + ad_c[pl.ds(g * 16, 16)]
            _spread_w16(W16, jnp.exp(_lrelu(a, 0.2)), g)
            return 0

        lax.fori_loop(0, C // 16, e1, 0)

        pltpu.async_copy(xp_hbm.at[src_c], G, sem).wait()
        _scale_rows(G, W16, C)
        pltpu.sync_copy(G, num_sh.at[dst_c], add=True)
        pltpu.sync_copy(W16, s_sh.at[dst_c], add=True)
        return 0

    lax.fori_loop(0, CH, chunk, 0)
    plsc.subcore_barrier()
    _writeback(num_sh, num_hbm, RPT_N, cid, sid)
    _writeback(s_sh, s_hbm, RPT_N, cid, sid)


_conv2_call = functools.partial(
    pl.kernel, _conv2_body,
    out_type=(jax.ShapeDtypeStruct((NC, NP, H), _f32),
              jax.ShapeDtypeStruct((NC, NP, 16), _f32)),
    mesh=_mesh,
    compiler_params=pltpu.CompilerParams(needs_layout_passes=False, use_tc_tiling_on_sc=False),
    scratch_types=[
        pltpu.VMEM_SHARED((NP, H), _f32),
        pltpu.VMEM_SHARED((NP, 16), _f32),
        pltpu.VMEM((C, H), _f32),
        pltpu.VMEM((C, 16), _f32),
        pltpu.VMEM((C,), jnp.int32),
        pltpu.VMEM((C,), jnp.int32),
        pltpu.VMEM((C,), _f32),
        pltpu.VMEM((C,), _f32),
        pltpu.SemaphoreType.DMA,
    ],
)


def _segsum_body(x_hbm, b_hbm, mol_hbm, mol_sh, X, b_c, sem):
    cid = lax.axis_index("c")
    sid = lax.axis_index("s")
    wid = cid * NS + sid

    _zero_rows(X, RPT_B, H)
    _zero_shared(X, mol_sh, RPT_B, sid)
    plsc.subcore_barrier()

    base = wid * NPW
    pltpu.sync_copy(x_hbm.at[pl.ds(base, NPW)], X)
    pltpu.sync_copy(b_hbm.at[pl.ds(base, NPW)], b_c)
    pltpu.sync_copy(X, mol_sh.at[b_c], add=True)

    plsc.subcore_barrier()
    _writeback(mol_sh, mol_hbm, RPT_B, cid, sid)


_segsum_call = functools.partial(
    pl.kernel, _segsum_body,
    out_type=jax.ShapeDtypeStruct((NC, BP, H), _f32),
    mesh=_mesh,
    compiler_params=pltpu.CompilerParams(needs_layout_passes=False, use_tc_tiling_on_sc=False),
    scratch_types=[
        pltpu.VMEM_SHARED((BP, H), _f32),
        pltpu.VMEM((NPW, H), _f32),
        pltpu.VMEM((NPW,), jnp.int32),
        pltpu.SemaphoreType.DMA,
    ],
)


def _readout_body(xs_hbm, as_hbm, ad_hbm, b_hbm,
                  num_hbm, s_hbm,
                  num_sh, s_sh,
                  X, W16, b_c, as_c, ad_v, sem):
    cid = lax.axis_index("c")
    sid = lax.axis_index("s")
    wid = cid * NS + sid

    _zero_rows(X, RPT_B, H)
    _zero_rows(W16, RPT_B, 16)
    pltpu.sync_copy(ad_hbm, ad_v)
    _zero_shared(X, num_sh, RPT_B, sid)
    _zero_shared(W16, s_sh, RPT_B, sid)
    plsc.subcore_barrier()

    base = wid * NPW
    pltpu.sync_copy(xs_hbm.at[pl.ds(base, NPW)], X)
    pltpu.sync_copy(b_hbm.at[pl.ds(base, NPW)], b_c)
    pltpu.sync_copy(as_hbm.at[pl.ds(base, NPW)], as_c)

    def e1(g, _):
        b16 = b_c[pl.ds(g * 16, 16)]
        a = as_c[pl.ds(g * 16, 16)] + plsc.load_gather(ad_v, [b16])
        _spread_w16(W16, jnp.exp(_lrelu(a, 0.2)), g)
        return 0

    lax.fori_loop(0, NPW // 16, e1, 0)
    _scale_rows(X, W16, NPW)
    pltpu.sync_copy(X, num_sh.at[b_c], add=True)
    pltpu.sync_copy(W16, s_sh.at[b_c], add=True)

    plsc.subcore_barrier()
    _writeback(num_sh, num_hbm, RPT_B, cid, sid)
    _writeback(s_sh, s_hbm, RPT_B, cid, sid)


_readout_call = functools.partial(
    pl.kernel, _readout_body,
    out_type=(jax.ShapeDtypeStruct((NC, BP, H), _f32),
              jax.ShapeDtypeStruct((NC, BP, 16), _f32)),
    mesh=_mesh,
    compiler_params=pltpu.CompilerParams(needs_layout_passes=False, use_tc_tiling_on_sc=False),
    scratch_types=[
        pltpu.VMEM_SHARED((BP, H), _f32),
        pltpu.VMEM_SHARED((BP, 16), _f32),
        pltpu.VMEM((NPW, H), _f32),
        pltpu.VMEM((NPW, 16), _f32),
        pltpu.VMEM((NPW,), jnp.int32),
        pltpu.VMEM((NPW,), _f32),
        pltpu.VMEM((BP,), _f32),
        pltpu.SemaphoreType.DMA,
    ],
)


def _elu(v):
    return jnp.where(v > 0, v, jnp.expm1(v))


def _gru(x, h, Wih, Whh, bih, bhh):
    gi = x @ Wih.T + bih
    gh = h @ Whh.T + bhh
    i_r, i_z, i_n = jnp.split(gi, 3, axis=-1)
    h_r, h_z, h_n = jnp.split(gh, 3, axis=-1)
    r = jax.nn.sigmoid(i_r + h_r)
    z = jax.nn.sigmoid(i_z + h_z)
    nn = jnp.tanh(i_n + r * h_n)
    return (1.0 - z) * nn + z * h


def kernel(x, edge_attr, edge_index, batch, W_in, b_in, att_l, att_r, W_edge,
           W_gout, b_gate, Wih0, Whh0, bih0, bhh0, Wa, att_src_a, att_dst_a,
           ba, Wih1, Whh1, bih1, bhh1, Wm, att_src_m, att_dst_m, bm,
           Wih2, Whh2, bih2, bhh2, W_out, b_out):
    src = edge_index[0]
    dst = edge_index[1]

    x1 = _lrelu(x @ W_in.T + b_in, 0.01)
    tl = x1 @ W_edge[:, :H].T
    eap = edge_attr @ W_edge[:, H:].T
    sr = x1 @ att_r
    gout = x1 @ W_gout.T

    num1p, s1p = _conv1_call()(tl, eap, sr, gout, att_l, src, dst)
    num1 = num1p[0, :N] + num1p[1, :N]
    s1 = s1p[0, :N, 0] + s1p[1, :N, 0]
    h1 = _elu(num1 / (s1[:, None] + EPS) + b_gate)
    x2 = jax.nn.relu(_gru(h1, x1, Wih0, Whh0, bih0, bhh0))

    xp = x2 @ Wa.T
    a_s = xp @ att_src_a
    a_d = xp @ att_dst_a
    num2p, s2p = _conv2_call()(xp, a_s, a_d, src, dst)
    num2 = num2p[0, :N] + num2p[1, :N]
    s2 = s2p[0, :N, 0] + s2p[1, :N, 0]
    h2 = _elu(num2 / (s2[:, None] + EPS) + ba)
    x3 = jax.nn.relu(_gru(h2, x2, Wih1, Whh1, bih1, bhh1))

    x3p = jnp.concatenate([x3, jnp.zeros((NP - N, H), _f32)], axis=0)
    batch_p = jnp.concatenate(
        [batch, jnp.full((NP - N,), B, jnp.int32)], axis=0)
    molp = _segsum_call()(x3p, batch_p)
    mol = jax.nn.relu(molp[0, :B] + molp[1, :B])

    xs = x3 @ Wm.T
    xsp = jnp.concatenate([xs, jnp.zeros((NP - N, H), _f32)], axis=0)
    a_sn = xsp @ att_src_m
    for _ in range(2):
        a_d_t = (mol @ Wm.T) @ att_dst_m
        a_d_p = jnp.concatenate([a_d_t, jnp.zeros((BP - B,), _f32)], axis=0)
        nump, sp = _readout_call()(xsp, a_sn, a_d_p, batch_p)
        num = nump[0, :B] + nump[1, :B]
        s = sp[0, :B, 0] + sp[1, :B, 0]
        hm = _elu(num / (s[:, None] + EPS) + bm)
        mol = jax.nn.relu(_gru(hm, mol, Wih2, Whh2, bih2, bhh2))

    return mol @ W_out.T + b_out


# speedup vs baseline: 16.9538x; 1.9386x over previous
import functools

import jax
import jax.numpy as jnp
from jax import lax
from jax.experimental import pallas as pl
from jax.experimental.pallas import tpu as pltpu
from jax.experimental.pallas import tpu_sc as plsc

H = 128
N = 10000
E = 320000
B = 512
NP = 10240
BP = 640
NC = 2
NS = 16
NW = NC * NS
EPW = E // NW
C = 80
CH = EPW // C
QUADS = (CH - 1) // 4
RPT_N = NP // NS
RPT_B = BP // NS
NPW = NP // NW
EPS = 1e-16

_mesh = plsc.VectorSubcoreMesh(core_axis_name="c", subcore_axis_name="s")
_f32 = jnp.float32
_params = pltpu.CompilerParams(needs_layout_passes=False,
                               use_tc_tiling_on_sc=False)


def _lrelu(v, s):
    return jnp.where(v >= 0, v, s * v)


def _zero_rows(buf, rows, cols):
    zv = jnp.zeros((16,), _f32)
    kk = cols // 16

    def body(i, _):
        r = i // kk
        k = i % kk
        buf[r, pl.ds(k * 16, 16)] = zv
        return 0

    lax.fori_loop(0, rows * kk, body, 0)


def _zero_shared(zsrc, sh, rows_per_tile, sid):
    zrows = zsrc.shape[0]
    base = sid * rows_per_tile
    done = 0
    while done < rows_per_tile:
        step = min(zrows, rows_per_tile - done)
        pltpu.sync_copy(zsrc.at[pl.ds(0, step)], sh.at[pl.ds(base + done, step)])
        done += step


def _writeback(sh, hbm, rows_per_tile, cid, sid):
    base = sid * rows_per_tile
    pltpu.sync_copy(sh.at[pl.ds(base, rows_per_tile)],
                    hbm.at[cid].at[pl.ds(base, rows_per_tile)])


def _vec_copy(src_ref, src_off, dst_ref, n):
    for i in range(n // 16):
        dst_ref[pl.ds(i * 16, 16)] = src_ref[pl.ds(src_off + i * 16, 16)]


def _scale_rows(Gbuf, W16, n_rows):
    def body(e, _):
        w = W16[e, pl.ds(0, 16)]
        for k in range(8):
            Gbuf[e, pl.ds(k * 16, 16)] = Gbuf[e, pl.ds(k * 16, 16)] * w
        return 0

    lax.fori_loop(0, n_rows, body, 0)


def _spread_w16(W16, ex16, g):
    for l in range(16):
        W16[g * 16 + l, pl.ds(0, 16)] = jnp.full((16,), ex16[l])


def _pipeline(do_chunk, issue, drain):
    issue(0, 0, 0)
    issue(1, 1, 1)

    def quad(Q, _):
        for q in range(4):
            k = 4 * Q + q
            b = q % 2
            if q < 2:
                @pl.when(Q > 0)
                def _():
                    drain(b)
            else:
                drain(b)
            do_chunk(k, b, q)
            if q == 3:
                @pl.when(Q < QUADS - 1)
                def _():
                    issue(k + 2, b, (q + 2) % 4)
            else:
                issue(k + 2, b, (q + 2) % 4)
        return 0

    lax.fori_loop(0, QUADS, quad, 0)
    drain(0)
    do_chunk(CH - 1, 0, 0)
    drain(0)
    drain(1)


def _c1a_body(tl_hbm, eap_hbm, sr_hbm, attl_hbm, src_hbm, dst_hbm,
              w_hbm, s_hbm,
              s_sh,
              A0, A1, B0, B1, W0, W1, sfull, dfull,
              sc0, sc1, dc0, dc1, dc2, dc3, ar0, ar1, attl_v,
              gA0, gA1, gB0, gB1, gR0, gR1, sW0, sW1, sS0, sS1):
    A = (A0, A1); Bb = (B0, B1); W16 = (W0, W1)
    srcc = (sc0, sc1); dcs = (dc0, dc1, dc2, dc3); arc = (ar0, ar1)
    semA = (gA0, gA1); semB = (gB0, gB1); semR = (gR0, gR1)
    semW = (sW0, sW1); semS = (sS0, sS1)

    cid = lax.axis_index("c")
    sid = lax.axis_index("s")
    wid = cid * NS + sid
    ebase = wid * EPW

    _zero_rows(W0, C, 16)
    _zero_shared(W0, s_sh, RPT_N, sid)
    pltpu.sync_copy(attl_hbm, attl_v)
    pltpu.sync_copy(src_hbm.at[pl.ds(ebase, EPW)], sfull)
    pltpu.sync_copy(dst_hbm.at[pl.ds(ebase, EPW)], dfull)
    plsc.subcore_barrier()

    def issue(k, b, d):
        off = k * C
        _vec_copy(sfull, off, srcc[b], C)
        _vec_copy(dfull, off, dcs[d], C)
        pltpu.async_copy(tl_hbm.at[srcc[b]], A[b], semA[b])
        pltpu.async_copy(eap_hbm.at[pl.ds(ebase + off, C)], Bb[b], semB[b])
        pltpu.async_copy(sr_hbm.at[dcs[d]], arc[b], semR[b])

    def drain(b):
        pltpu.make_async_copy(W16[b], w_hbm.at[pl.ds(0, C)], semW[b]).wait()
        pltpu.make_async_copy(W16[b], s_sh.at[dcs[0]], semS[b]).wait()

    def do_chunk(k, b, d):
        pltpu.make_async_copy(tl_hbm.at[srcc[b]], A[b], semA[b]).wait()
        pltpu.make_async_copy(eap_hbm.at[pl.ds(0, C)], Bb[b], semB[b]).wait()
        pltpu.make_async_copy(sr_hbm.at[dcs[0]], arc[b], semR[b]).wait()

        def e1(g, _):
            ar16 = arc[b][pl.ds(g * 16, 16)]
            for l in range(16):
                e = g * 16 + l
                acc = jnp.zeros((16,), _f32)
                for kk in range(8):
                    v = A[b][e, pl.ds(kk * 16, 16)] + Bb[b][e, pl.ds(kk * 16, 16)]
                    v = _lrelu(v, 0.01)
                    acc = acc + v * attl_v[pl.ds(kk * 16, 16)]
                aj = jnp.sum(acc)
                al = _lrelu(aj + ar16[l], 0.01)
                W16[b][e, pl.ds(0, 16)] = jnp.exp(jnp.full((16,), al))
            return 0

        lax.fori_loop(0, C // 16, e1, 0)
        pltpu.async_copy(W16[b], w_hbm.at[pl.ds(ebase + k * C, C)], semW[b])
        pltpu.async_copy(W16[b], s_sh.at[dcs[d]], semS[b], add=True)

    _pipeline(do_chunk, issue, drain)
    plsc.subcore_barrier()
    _writeback(s_sh, s_hbm, RPT_N, cid, sid)


_c1a_call = functools.partial(
    pl.kernel, _c1a_body,
    out_type=(jax.ShapeDtypeStruct((E, 16), _f32),
              jax.ShapeDtypeStruct((NC, NP, 16), _f32)),
    mesh=_mesh,
    compiler_params=_params,
    scratch_types=(
        [pltpu.VMEM_SHARED((NP, 16), _f32)]
        + [pltpu.VMEM((C, H), _f32)] * 4
        + [pltpu.VMEM((C, 16), _f32)] * 2
        + [pltpu.VMEM((EPW,), jnp.int32)] * 2
        + [pltpu.VMEM((C,), jnp.int32)] * 6
        + [pltpu.VMEM((C,), _f32)] * 2
        + [pltpu.VMEM((H,), _f32)]
        + [pltpu.SemaphoreType.DMA] * 10
    ),
)


def _c1b_body(rows_hbm, w_hbm, src_hbm, dst_hbm,
              num_hbm,
              num_sh,
              G0, G1, W0, W1, sfull, dfull,
              sc0, sc1, dc0, dc1, dc2, dc3,
              gG0, gG1, gV0, gV1, sN0, sN1):
    G = (G0, G1); W16 = (W0, W1)
    srcc = (sc0, sc1); dcs = (dc0, dc1, dc2, dc3)
    semG = (gG0, gG1); semV = (gV0, gV1); semN = (sN0, sN1)

    cid = lax.axis_index("c")
    sid = lax.axis_index("s")
    wid = cid * NS + sid
    ebase = wid * EPW

    _zero_rows(G0, C, H)
    _zero_shared(G0, num_sh, RPT_N, sid)
    pltpu.sync_copy(src_hbm.at[pl.ds(ebase, EPW)], sfull)
    pltpu.sync_copy(dst_hbm.at[pl.ds(ebase, EPW)], dfull)
    plsc.subcore_barrier()

    def issue(k, b, d):
        off = k * C
        _vec_copy(sfull, off, srcc[b], C)
        _vec_copy(dfull, off, dcs[d], C)
        pltpu.async_copy(rows_hbm.at[srcc[b]], G[b], semG[b])
        pltpu.async_copy(w_hbm.at[pl.ds(ebase + off, C)], W16[b], semV[b])

    def drain(b):
        pltpu.make_async_copy(G[b], num_sh.at[dcs[0]], semN[b]).wait()

    def do_chunk(k, b, d):
        pltpu.make_async_copy(rows_hbm.at[srcc[b]], G[b], semG[b]).wait()
        pltpu.make_async_copy(w_hbm.at[pl.ds(0, C)], W16[b], semV[b]).wait()
        _scale_rows(G[b], W16[b], C)
        pltpu.async_copy(G[b], num_sh.at[dcs[d]], semN[b], add=True)

    _pipeline(do_chunk, issue, drain)
    plsc.subcore_barrier()
    _writeback(num_sh, num_hbm, RPT_N, cid, sid)


_c1b_call = functools.partial(
    pl.kernel, _c1b_body,
    out_type=jax.ShapeDtypeStruct((NC, NP, H), _f32),
    mesh=_mesh,
    compiler_params=_params,
    scratch_types=(
        [pltpu.VMEM_SHARED((NP, H), _f32)]
        + [pltpu.VMEM((C, H), _f32)] * 2
        + [pltpu.VMEM((C, 16), _f32)] * 2
        + [pltpu.VMEM((EPW,), jnp.int32)] * 2
        + [pltpu.VMEM((C,), jnp.int32)] * 6
        + [pltpu.SemaphoreType.DMA] * 6
    ),
)


def _conv2_body(xp_hbm, as_hbm, ad_hbm, src_hbm, dst_hbm,
                num_hbm, s_hbm,
                num_sh, s_sh,
                G0, G1, W0, W1, dfull,
                sc0, sc1, dc0, dc1, dc2, dc3, as0, as1, ad0, ad1,
                gG0, gG1, gA0, gA1, gD0, gD1, sN0, sN1, sS0, sS1):
    G = (G0, G1); W16 = (W0, W1)
    srcc = (sc0, sc1); dcs = (dc0, dc1, dc2, dc3)
    asc = (as0, as1); adc = (ad0, ad1)
    semG = (gG0, gG1); semA = (gA0, gA1); semD = (gD0, gD1)
    semN = (sN0, sN1); semS = (sS0, sS1)

    cid = lax.axis_index("c")
    sid = lax.axis_index("s")
    wid = cid * NS + sid
    ebase = wid * EPW

    _zero_rows(G0, C, H)
    _zero_rows(W0, C, 16)
    _zero_shared(G0, num_sh, RPT_N, sid)
    _zero_shared(W0, s_sh, RPT_N, sid)
    pltpu.sync_copy(dst_hbm.at[pl.ds(ebase, EPW)], dfull)
    plsc.subcore_barrier()

    def issue(k, b, d):
        off = k * C
        pltpu.sync_copy(src_hbm.at[pl.ds(ebase + off, C)], srcc[b])
        _vec_copy(dfull, off, dcs[d], C)
        pltpu.async_copy(xp_hbm.at[srcc[b]], G[b], semG[b])
        pltpu.async_copy(as_hbm.at[srcc[b]], asc[b], semA[b])
        pltpu.async_copy(ad_hbm.at[dcs[d]], adc[b], semD[b])

    def drain(b):
        pltpu.make_async_copy(G[b], num_sh.at[dcs[0]], semN[b]).wait()
        pltpu.make_async_copy(W16[b], s_sh.at[dcs[0]], semS[b]).wait()

    def do_chunk(k, b, d):
        pltpu.make_async_copy(as_hbm.at[srcc[b]], asc[b], semA[b]).wait()
        pltpu.make_async_copy(ad_hbm.at[dcs[0]], adc[b], semD[b]).wait()

        def e1(g, _):
            a = asc[b][pl.ds(g * 16, 16)] + adc[b][pl.ds(g * 16, 16)]
            _spread_w16(W16[b], jnp.exp(_lrelu(a, 0.2)), g)
            return 0

        lax.fori_loop(0, C // 16, e1, 0)
        pltpu.make_async_copy(xp_hbm.at[srcc[b]], G[b], semG[b]).wait()
        _scale_rows(G[b], W16[b], C)
        pltpu.async_copy(G[b], num_sh.at[dcs[d]], semN[b], add=True)
        pltpu.async_copy(W16[b], s_sh.at[dcs[d]], semS[b], add=True)

    _pipeline(do_chunk, issue, drain)
    plsc.subcore_barrier()
    _writeback(num_sh, num_hbm, RPT_N, cid, sid)
    _writeback(s_sh, s_hbm, RPT_N, cid, sid)


_conv2_call = functools.partial(
    pl.kernel, _conv2_body,
    out_type=(jax.ShapeDtypeStruct((NC, NP, H), _f32),
              jax.ShapeDtypeStruct((NC, NP, 16), _f32)),
    mesh=_mesh,
    compiler_params=_params,
    scratch_types=(
        [pltpu.VMEM_SHARED((NP, H), _f32),
         pltpu.VMEM_SHARED((NP, 16), _f32)]
        + [pltpu.VMEM((C, H), _f32)] * 2
        + [pltpu.VMEM((C, 16), _f32)] * 2
        + [pltpu.VMEM((EPW,), jnp.int32)]
        + [pltpu.VMEM((C,), jnp.int32)] * 6
        + [pltpu.VMEM((C,), _f32)] * 4
        + [pltpu.SemaphoreType.DMA] * 10
    ),
)


def _segsum_body(x_hbm, b_hbm, mol_hbm, mol_sh, X, b_c, sem):
    cid = lax.axis_index("c")
    sid = lax.axis_index("s")
    wid = cid * NS + sid

    _zero_rows(X, RPT_B, H)
    _zero_shared(X, mol_sh, RPT_B, sid)
    plsc.subcore_barrier()

    base = wid * NPW
    pltpu.sync_copy(x_hbm.at[pl.ds(base, NPW)], X)
    pltpu.sync_copy(b_hbm.at[pl.ds(base, NPW)], b_c)
    pltpu.sync_copy(X, mol_sh.at[b_c], add=True)

    plsc.subcore_barrier()
    _writeback(mol_sh, mol_hbm, RPT_B, cid, sid)


_segsum_call = functools.partial(
    pl.kernel, _segsum_body,
    out_type=jax.ShapeDtypeStruct((NC, BP, H), _f32),
    mesh=_mesh,
    compiler_params=_params,
    scratch_types=[
        pltpu.VMEM_SHARED((BP, H), _f32),
        pltpu.VMEM((NPW, H), _f32),
        pltpu.VMEM((NPW,), jnp.int32),
        pltpu.SemaphoreType.DMA,
    ],
)


def _readout_body(xs_hbm, as_hbm, ad_hbm, b_hbm,
                  num_hbm, s_hbm,
                  num_sh, s_sh,
                  X, W16, b_c, as_c, ad_v, sem):
    cid = lax.axis_index("c")
    sid = lax.axis_index("s")
    wid = cid * NS + sid

    _zero_rows(X, RPT_B, H)
    _zero_rows(W16, RPT_B, 16)
    pltpu.sync_copy(ad_hbm, ad_v)
    _zero_shared(X, num_sh, RPT_B, sid)
    _zero_shared(W16, s_sh, RPT_B, sid)
    plsc.subcore_barrier()

    base = wid * NPW
    pltpu.sync_copy(xs_hbm.at[pl.ds(base, NPW)], X)
    pltpu.sync_copy(b_hbm.at[pl.ds(base, NPW)], b_c)
    pltpu.sync_copy(as_hbm.at[pl.ds(base, NPW)], as_c)

    def e1(g, _):
        b16 = b_c[pl.ds(g * 16, 16)]
        a = as_c[pl.ds(g * 16, 16)] + plsc.load_gather(ad_v, [b16])
        _spread_w16(W16, jnp.exp(_lrelu(a, 0.2)), g)
        return 0

    lax.fori_loop(0, NPW // 16, e1, 0)
    _scale_rows(X, W16, NPW)
    pltpu.sync_copy(X, num_sh.at[b_c], add=True)
    pltpu.sync_copy(W16, s_sh.at[b_c], add=True)

    plsc.subcore_barrier()
    _writeback(num_sh, num_hbm, RPT_B, cid, sid)
    _writeback(s_sh, s_hbm, RPT_B, cid, sid)


_readout_call = functools.partial(
    pl.kernel, _readout_body,
    out_type=(jax.ShapeDtypeStruct((NC, BP, H), _f32),
              jax.ShapeDtypeStruct((NC, BP, 16), _f32)),
    mesh=_mesh,
    compiler_params=_params,
    scratch_types=[
        pltpu.VMEM_SHARED((BP, H), _f32),
        pltpu.VMEM_SHARED((BP, 16), _f32),
        pltpu.VMEM((NPW, H), _f32),
        pltpu.VMEM((NPW, 16), _f32),
        pltpu.VMEM((NPW,), jnp.int32),
        pltpu.VMEM((NPW,), _f32),
        pltpu.VMEM((BP,), _f32),
        pltpu.SemaphoreType.DMA,
    ],
)


def _elu(v):
    return jnp.where(v > 0, v, jnp.expm1(v))


def _gru(x, h, Wih, Whh, bih, bhh):
    gi = x @ Wih.T + bih
    gh = h @ Whh.T + bhh
    i_r, i_z, i_n = jnp.split(gi, 3, axis=-1)
    h_r, h_z, h_n = jnp.split(gh, 3, axis=-1)
    r = jax.nn.sigmoid(i_r + h_r)
    z = jax.nn.sigmoid(i_z + h_z)
    nn = jnp.tanh(i_n + r * h_n)
    return (1.0 - z) * nn + z * h


def kernel(x, edge_attr, edge_index, batch, W_in, b_in, att_l, att_r, W_edge,
           W_gout, b_gate, Wih0, Whh0, bih0, bhh0, Wa, att_src_a, att_dst_a,
           ba, Wih1, Whh1, bih1, bhh1, Wm, att_src_m, att_dst_m, bm,
           Wih2, Whh2, bih2, bhh2, W_out, b_out):
    src = edge_index[0]
    dst = edge_index[1]

    x1 = _lrelu(x @ W_in.T + b_in, 0.01)
    tl = x1 @ W_edge[:, :H].T
    eap = edge_attr @ W_edge[:, H:].T
    sr = x1 @ att_r
    gout = x1 @ W_gout.T

    w16, s1p = _c1a_call()(tl, eap, sr, att_l, src, dst)
    num1p = _c1b_call()(gout, w16, src, dst)
    num1 = num1p[0, :N] + num1p[1, :N]
    s1 = s1p[0, :N, 0] + s1p[1, :N, 0]
    h1 = _elu(num1 / (s1[:, None] + EPS) + b_gate)
    x2 = jax.nn.relu(_gru(h1, x1, Wih0, Whh0, bih0, bhh0))

    xp = x2 @ Wa.T
    a_s = xp @ att_src_a
    a_d = xp @ att_dst_a
    num2p, s2p = _conv2_call()(xp, a_s, a_d, src, dst)
    num2 = num2p[0, :N] + num2p[1, :N]
    s2 = s2p[0, :N, 0] + s2p[1, :N, 0]
    h2 = _elu(num2 / (s2[:, None] + EPS) + ba)
    x3 = jax.nn.relu(_gru(h2, x2, Wih1, Whh1, bih1, bhh1))

    x3p = jnp.concatenate([x3, jnp.zeros((NP - N, H), _f32)], axis=0)
    batch_p = jnp.concatenate(
        [batch, jnp.full((NP - N,), B, jnp.int32)], axis=0)
    molp = _segsum_call()(x3p, batch_p)
    mol = jax.nn.relu(molp[0, :B] + molp[1, :B])

    xs = x3 @ Wm.T
    xsp = jnp.concatenate([xs, jnp.zeros((NP - N, H), _f32)], axis=0)
    a_sn = xsp @ att_src_m
    for _ in range(2):
        a_d_t = (mol @ Wm.T) @ att_dst_m
        a_d_p = jnp.concatenate([a_d_t, jnp.zeros((BP - B,), _f32)], axis=0)
        nump, sp = _readout_call()(xsp, a_sn, a_d_p, batch_p)
        num = nump[0, :B] + nump[1, :B]
        s = sp[0, :B, 0] + sp[1, :B, 0]
        hm = _elu(num / (s[:, None] + EPS) + bm)
        mol = jax.nn.relu(_gru(hm, mol, Wih2, Whh2, bih2, bhh2))

    return mol @ W_out.T + b_out
